# baseline (speedup 1.0000x reference)
import functools

import jax
import jax.numpy as jnp
from jax import lax
from jax.experimental import pallas as pl
from jax.experimental.pallas import tpu as pltpu
from jax.experimental.pallas import tpu_sc as plsc

N, M, F, D_E, H = 10000, 32, 128, 16, 128
BN_BLK = 400
NBLK = N // BN_BLK

def _q_kernel(atom_ref, w1b_ref, q_ref):
    q = jnp.dot(atom_ref[...], w1b_ref[...],
                preferred_element_type=jnp.float32)
    rid = (lax.broadcasted_iota(jnp.int32, (BN_BLK, H), 0)
           + pl.program_id(0) * BN_BLK)
    q_ref[...] = jnp.where(rid == 0, -1e9, q)


def _compute_q(atom, w1b):
    return pl.pallas_call(
        _q_kernel,
        grid=(NBLK,),
        in_specs=[
            pl.BlockSpec((BN_BLK, F), lambda i: (i, 0)),
            pl.BlockSpec((F, H), lambda i: (0, 0)),
        ],
        out_specs=pl.BlockSpec((BN_BLK, H), lambda i: (i, 0)),
        out_shape=jax.ShapeDtypeStruct((N, H), jnp.float32),
    )(atom, w1b)


_NC, _NS = 2, 16
_NW = _NC * _NS
_CHUNK = 400
K_SL = 5
N_SL = N // K_SL
_NCHUNK = N_SL // _CHUNK
NBLK_SL = N_SL // BN_BLK


def _make_gather_body(k):
    def body(q_hbm, idx_hbm, out_hbm, pos_v, idx_v, rows_v, sem):
        wid = lax.axis_index("s") * _NC + lax.axis_index("c")
        lanes = lax.iota(jnp.int32, 16)

        def step(c, _):
            off = c * _CHUNK
            base_i = k * N_SL + off
            for t in range(_CHUNK // 16):
                pos_v[pl.ds(16 * t, 16)] = (base_i + 16 * t + lanes) * M + wid
            pltpu.async_copy(idx_hbm.at[pos_v], idx_v, sem).wait()
            pltpu.async_copy(q_hbm.at[idx_v], rows_v, sem).wait()
            pltpu.sync_copy(rows_v, out_hbm.at[pl.ds(wid * N_SL + off,
                                                     _CHUNK)])
            return ()

        lax.fori_loop(0, _NCHUNK, step, (), unroll=False)
    return body


def _gather_rows(q, idx_flat, k):
    mesh = plsc.VectorSubcoreMesh(core_axis_name="c", subcore_axis_name="s")
    fn = functools.partial(
        pl.kernel, mesh=mesh,
        out_type=jax.ShapeDtypeStruct((M * N_SL, H), jnp.float32),
        scratch_types=[
            pltpu.VMEM((_CHUNK,), jnp.int32),
            pltpu.VMEM((_CHUNK,), jnp.int32),
            pltpu.VMEM((_CHUNK, H), jnp.float32),
            pltpu.SemaphoreType.DMA,
        ],
    )(_make_gather_body(k))
    return fn(q, idx_flat)


_LOG2E = 1.4426950408889634


def _softplus(x):
    t = lax.exp2(-jnp.abs(x) * _LOG2E)
    return jnp.maximum(x, 0.0) + jnp.log(1.0 + t)


def _msg_kernel(atom_ref, qg_ref, nbr_ref, idxf_ref, w1a_ref, w1c_ref,
                b1_ref, w2_ref, b2_ref, out_ref, psum_ref, psumsq_ref):
    atom = atom_ref[...]
    p = jnp.dot(atom, w1a_ref[...],
                preferred_element_type=jnp.float32) + b1_ref[...]
    w1c = w1c_ref[...]
    acc = None
    for j in range(M):
        ej = jnp.dot(nbr_ref[:, j * D_E:(j + 1) * D_E], w1c,
                     preferred_element_type=jnp.float32)
        hj = _softplus(qg_ref[j] + ej + p)
        acc = hj if acc is None else acc + hj
    cnt = jnp.sum(jnp.where(idxf_ref[...] != 0.0, 1.0, 0.0),
                  axis=1, keepdims=True)
    msg = (jnp.dot(acc, w2_ref[...], preferred_element_type=jnp.float32)
           + cnt * b2_ref[...])
    out_pre = atom + msg
    out_ref[...] = out_pre
    psum_ref[...] = jnp.sum(out_pre, axis=0, keepdims=True)[None]
    psumsq_ref[...] = jnp.sum(out_pre * out_pre, axis=0, keepdims=True)[None]


def _compute_msg(atom, qg3, nbr2, idxf2, w1a, w1c, b1r, w2, b2r, k):
    blk0 = k * NBLK_SL
    return pl.pallas_call(
        _msg_kernel,
        grid=(NBLK_SL,),
        in_specs=[
            pl.BlockSpec((BN_BLK, F), lambda i: (blk0 + i, 0)),
            pl.BlockSpec((M, BN_BLK, H), lambda i: (0, i, 0)),
            pl.BlockSpec((BN_BLK, M * D_E), lambda i: (blk0 + i, 0)),
            pl.BlockSpec((BN_BLK, M), lambda i: (blk0 + i, 0)),
            pl.BlockSpec((F, H), lambda i: (0, 0)),
            pl.BlockSpec((D_E, H), lambda i: (0, 0)),
            pl.BlockSpec((1, H), lambda i: (0, 0)),
            pl.BlockSpec((H, F), lambda i: (0, 0)),
            pl.BlockSpec((1, F), lambda i: (0, 0)),
        ],
        out_specs=[
            pl.BlockSpec((BN_BLK, F), lambda i: (i, 0)),
            pl.BlockSpec((1, 1, F), lambda i: (i, 0, 0)),
            pl.BlockSpec((1, 1, F), lambda i: (i, 0, 0)),
        ],
        out_shape=[
            jax.ShapeDtypeStruct((N_SL, F), jnp.float32),
            jax.ShapeDtypeStruct((NBLK_SL, 1, F), jnp.float32),
            jax.ShapeDtypeStruct((NBLK_SL, 1, F), jnp.float32),
        ],
    )(atom, qg3, nbr2, idxf2, w1a, w1c, b1r, w2, b2r)


def _bn_kernel(x_ref, psum_ref, psumsq_ref, gamma_ref, beta_ref, out_ref):
    mean = jnp.sum(psum_ref[...], axis=0) / N
    ex2 = jnp.sum(psumsq_ref[...], axis=0) / N
    var = ex2 - mean * mean
    inv = lax.rsqrt(var + 1e-5)
    y = (x_ref[...] - mean) * (inv * gamma_ref[...]) + beta_ref[...]
    out_ref[...] = _softplus(y)


def _apply_bn(x, psum, psumsq, gammar, betar):
    return pl.pallas_call(
        _bn_kernel,
        grid=(NBLK,),
        in_specs=[
            pl.BlockSpec((BN_BLK, F), lambda i: (i, 0)),
            pl.BlockSpec((NBLK, 1, F), lambda i: (0, 0, 0)),
            pl.BlockSpec((NBLK, 1, F), lambda i: (0, 0, 0)),
            pl.BlockSpec((1, F), lambda i: (0, 0)),
            pl.BlockSpec((1, F), lambda i: (0, 0)),
        ],
        out_specs=pl.BlockSpec((BN_BLK, F), lambda i: (i, 0)),
        out_shape=jax.ShapeDtypeStruct((N, F), jnp.float32),
    )(x, psum, psumsq, gammar, betar)


def kernel(atom_in_fea, nbr_fea, nbr_fea_idx, W1, b1, W2, b2,
           bn_gamma, bn_beta):
    w1a = W1[:F]
    w1b = W1[F:2 * F]
    w1c = W1[2 * F:]
    b1r = b1.reshape(1, H)
    b2r = b2.reshape(1, F)
    gammar = bn_gamma.reshape(1, F)
    betar = bn_beta.reshape(1, F)
    nbr2 = nbr_fea.reshape(N, M * D_E)
    idxf2 = nbr_fea_idx.astype(jnp.float32)

    idx_flat = nbr_fea_idx.reshape(N * M)
    q = _compute_q(atom_in_fea, w1b)

    outs, psums, psumsqs = [], [], []
    for k in range(K_SL):
        qg3 = _gather_rows(q, idx_flat, k).reshape(M, N_SL, H)
        o, ps, pq = _compute_msg(
            atom_in_fea, qg3, nbr2, idxf2, w1a, w1c, b1r, W2, b2r, k)
        outs.append(o)
        psums.append(ps)
        psumsqs.append(pq)

    out_pre = jnp.concatenate(outs, axis=0)
    psum = jnp.concatenate(psums, axis=0)
    psumsq = jnp.concatenate(psumsqs, axis=0)
    return _apply_bn(out_pre, psum, psumsq, gammar, betar)


# speedup vs baseline: 1.1278x; 1.1278x over previous
import functools

import jax
import jax.numpy as jnp
from jax import lax
from jax.experimental import pallas as pl
from jax.experimental.pallas import tpu as pltpu
from jax.experimental.pallas import tpu_sc as plsc

N, M, F, D_E, H = 10000, 32, 128, 16, 128
BN_BLK = 400
NBLK = N // BN_BLK

def _q_kernel(atom_ref, w1b_ref, q_ref):
    q = jnp.dot(atom_ref[...], w1b_ref[...],
                preferred_element_type=jnp.float32)
    rid = (lax.broadcasted_iota(jnp.int32, (BN_BLK, H), 0)
           + pl.program_id(0) * BN_BLK)
    q_ref[...] = jnp.where(rid == 0, -1e9, q)


def _compute_q(atom, w1b):
    return pl.pallas_call(
        _q_kernel,
        grid=(NBLK,),
        in_specs=[
            pl.BlockSpec((BN_BLK, F), lambda i: (i, 0)),
            pl.BlockSpec((F, H), lambda i: (0, 0)),
        ],
        out_specs=pl.BlockSpec((BN_BLK, H), lambda i: (i, 0)),
        out_shape=jax.ShapeDtypeStruct((N, H), jnp.float32),
    )(atom, w1b)


_NC, _NS = 2, 16
_NW = _NC * _NS
_CHUNK = 400
K_SL = 5
N_SL = N // K_SL
_NCHUNK = N_SL // _CHUNK
NBLK_SL = N_SL // BN_BLK


def _make_gather_body(k):
    def body(q_hbm, idx_hbm, out_hbm, idx_v, rows0, rows1, sem0, sem1):
        wid = lax.axis_index("s") * _NC + lax.axis_index("c")
        src0 = wid * N + k * N_SL
        dst0 = wid * N_SL
        pltpu.sync_copy(idx_hbm.at[pl.ds(src0, N_SL)], idx_v)
        rows = (rows0, rows1)
        sems = (sem0, sem1)

        def start(c):
            return pltpu.async_copy(
                q_hbm.at[idx_v.at[pl.ds(c * _CHUNK, _CHUNK)]],
                rows[c % 2], sems[c % 2])

        handles = {0: start(0), 1: start(1)}
        for c in range(_NCHUNK):
            handles[c % 2].wait()
            pltpu.sync_copy(rows[c % 2],
                            out_hbm.at[pl.ds(dst0 + c * _CHUNK, _CHUNK)])
            if c + 2 < _NCHUNK:
                handles[c % 2] = start(c + 2)
    return body


def _gather_rows(q, idx_flat_t, k):
    mesh = plsc.VectorSubcoreMesh(core_axis_name="c", subcore_axis_name="s")
    fn = functools.partial(
        pl.kernel, mesh=mesh,
        out_type=jax.ShapeDtypeStruct((M * N_SL, H), jnp.float32),
        scratch_types=[
            pltpu.VMEM((N_SL,), jnp.int32),
            pltpu.VMEM((_CHUNK, H), jnp.float32),
            pltpu.VMEM((_CHUNK, H), jnp.float32),
            pltpu.SemaphoreType.DMA,
            pltpu.SemaphoreType.DMA,
        ],
    )(_make_gather_body(k))
    return fn(q, idx_flat_t)


_LOG2E = 1.4426950408889634


def _softplus(x):
    t = lax.exp2(-jnp.abs(x) * _LOG2E)
    return jnp.maximum(x, 0.0) + jnp.log(1.0 + t)


def _msg_kernel(atom_ref, qg_ref, nbr_ref, idxf_ref, w1a_ref, w1c_ref,
                b1_ref, w2_ref, b2_ref, out_ref, psum_ref, psumsq_ref):
    atom = atom_ref[...]
    p = jnp.dot(atom, w1a_ref[...],
                preferred_element_type=jnp.float32) + b1_ref[...]
    w1c = w1c_ref[...]
    acc = None
    for j in range(M):
        ej = jnp.dot(nbr_ref[:, j * D_E:(j + 1) * D_E], w1c,
                     preferred_element_type=jnp.float32)
        hj = _softplus(qg_ref[j] + ej + p)
        acc = hj if acc is None else acc + hj
    cnt = jnp.sum(jnp.where(idxf_ref[...] != 0.0, 1.0, 0.0),
                  axis=1, keepdims=True)
    msg = (jnp.dot(acc, w2_ref[...], preferred_element_type=jnp.float32)
           + cnt * b2_ref[...])
    out_pre = atom + msg
    out_ref[...] = out_pre
    psum_ref[...] = jnp.sum(out_pre, axis=0, keepdims=True)[None]
    psumsq_ref[...] = jnp.sum(out_pre * out_pre, axis=0, keepdims=True)[None]


def _compute_msg(atom, qg3, nbr2, idxf2, w1a, w1c, b1r, w2, b2r, k):
    blk0 = k * NBLK_SL
    return pl.pallas_call(
        _msg_kernel,
        grid=(NBLK_SL,),
        in_specs=[
            pl.BlockSpec((BN_BLK, F), lambda i: (blk0 + i, 0)),
            pl.BlockSpec((M, BN_BLK, H), lambda i: (0, i, 0)),
            pl.BlockSpec((BN_BLK, M * D_E), lambda i: (blk0 + i, 0)),
            pl.BlockSpec((BN_BLK, M), lambda i: (blk0 + i, 0)),
            pl.BlockSpec((F, H), lambda i: (0, 0)),
            pl.BlockSpec((D_E, H), lambda i: (0, 0)),
            pl.BlockSpec((1, H), lambda i: (0, 0)),
            pl.BlockSpec((H, F), lambda i: (0, 0)),
            pl.BlockSpec((1, F), lambda i: (0, 0)),
        ],
        out_specs=[
            pl.BlockSpec((BN_BLK, F), lambda i: (i, 0)),
            pl.BlockSpec((1, 1, F), lambda i: (i, 0, 0)),
            pl.BlockSpec((1, 1, F), lambda i: (i, 0, 0)),
        ],
        out_shape=[
            jax.ShapeDtypeStruct((N_SL, F), jnp.float32),
            jax.ShapeDtypeStruct((NBLK_SL, 1, F), jnp.float32),
            jax.ShapeDtypeStruct((NBLK_SL, 1, F), jnp.float32),
        ],
    )(atom, qg3, nbr2, idxf2, w1a, w1c, b1r, w2, b2r)


def _bn_kernel(x_ref, psum_ref, psumsq_ref, gamma_ref, beta_ref, out_ref):
    mean = jnp.sum(psum_ref[...], axis=0) / N
    ex2 = jnp.sum(psumsq_ref[...], axis=0) / N
    var = ex2 - mean * mean
    inv = lax.rsqrt(var + 1e-5)
    y = (x_ref[...] - mean) * (inv * gamma_ref[...]) + beta_ref[...]
    out_ref[...] = _softplus(y)


def _apply_bn(x, psum, psumsq, gammar, betar):
    return pl.pallas_call(
        _bn_kernel,
        grid=(NBLK,),
        in_specs=[
            pl.BlockSpec((BN_BLK, F), lambda i: (i, 0)),
            pl.BlockSpec((NBLK, 1, F), lambda i: (0, 0, 0)),
            pl.BlockSpec((NBLK, 1, F), lambda i: (0, 0, 0)),
            pl.BlockSpec((1, F), lambda i: (0, 0)),
            pl.BlockSpec((1, F), lambda i: (0, 0)),
        ],
        out_specs=pl.BlockSpec((BN_BLK, F), lambda i: (i, 0)),
        out_shape=jax.ShapeDtypeStruct((N, F), jnp.float32),
    )(x, psum, psumsq, gammar, betar)


def kernel(atom_in_fea, nbr_fea, nbr_fea_idx, W1, b1, W2, b2,
           bn_gamma, bn_beta):
    w1a = W1[:F]
    w1b = W1[F:2 * F]
    w1c = W1[2 * F:]
    b1r = b1.reshape(1, H)
    b2r = b2.reshape(1, F)
    gammar = bn_gamma.reshape(1, F)
    betar = bn_beta.reshape(1, F)
    nbr2 = nbr_fea.reshape(N, M * D_E)
    idxf2 = nbr_fea_idx.astype(jnp.float32)

    idx_flat_t = nbr_fea_idx.T.reshape(M * N)
    q = _compute_q(atom_in_fea, w1b)

    outs, psums, psumsqs = [], [], []
    for k in range(K_SL):
        qg3 = _gather_rows(q, idx_flat_t, k).reshape(M, N_SL, H)
        o, ps, pq = _compute_msg(
            atom_in_fea, qg3, nbr2, idxf2, w1a, w1c, b1r, W2, b2r, k)
        outs.append(o)
        psums.append(ps)
        psumsqs.append(pq)

    out_pre = jnp.concatenate(outs, axis=0)
    psum = jnp.concatenate(psums, axis=0)
    psumsq = jnp.concatenate(psumsqs, axis=0)
    return _apply_bn(out_pre, psum, psumsq, gammar, betar)
